# baseline (device time: 15689 ns/iter reference)
import jax
import jax.numpy as jnp
from jax import lax
from jax.experimental import pallas as pl
from jax.experimental.pallas import tpu as pltpu

N_DEV = 4
B, SQ, SKV, HQ, DH = 2, 128, 512, 4, 64
D_MODEL = 512
D_QK = HQ * DH
SKV_PER = SKV // N_DEV
PRODUCERS = ((0, 0), (2, 1))


def kernel(x, Wq, K_ext, V_ext, Wo):
    K2 = K_ext.reshape(B, SKV_PER, D_QK)
    V2 = V_ext.reshape(B, SKV_PER, D_QK)

    def body(x_ref, wq_ref, k_ref, v_ref, wo_ref, out_ref,
             psend, lsend, pbuf, lbuf, send_sems, recv_sems, local_sems):
        my_pos = lax.axis_index("i")

        barrier_sem = pltpu.get_barrier_semaphore()
        for o in range(1, N_DEV):
            pl.semaphore_signal(
                barrier_sem, inc=1,
                device_id=((my_pos + o) % N_DEV,),
                device_id_type=pl.DeviceIdType.MESH,
            )
        pl.semaphore_wait(barrier_sem, N_DEV - 1)

        row = lax.broadcasted_iota(jnp.int32, (SQ, SKV_PER), 0)
        col = lax.broadcasted_iota(jnp.int32, (SQ, SKV_PER), 1)
        mask = (row // 64) == (col // 64)

        for producer, slot in PRODUCERS:
            @pl.when(my_pos == producer)
            def _(producer=producer, slot=slot):
                wq = wq_ref[...].astype(jnp.bfloat16)
                for b in range(B):
                    xb = x_ref[b].astype(jnp.bfloat16)
                    qb = lax.dot_general(
                        xb, wq, (((1,), (0,)), ((), ())),
                        preferred_element_type=jnp.float32,
                    )
                    ctx_heads = []
                    l_cols = []
                    for h in range(HQ):
                        hs = slice(h * DH, (h + 1) * DH)
                        q_h = qb[:, hs].astype(jnp.bfloat16)
                        k_h = k_ref[b][:, hs].astype(jnp.bfloat16)
                        scores = lax.dot_general(
                            q_h, k_h, (((1,), (1,)), ((), ())),
                            preferred_element_type=jnp.float32,
                        ) * 0.125
                        w = jnp.where(mask, jnp.exp(scores), 0.0)
                        l_cols.append(jnp.sum(w, axis=1, keepdims=True))
                        v_h = v_ref[b][:, hs].astype(jnp.bfloat16)
                        ctx_heads.append(lax.dot_general(
                            w.astype(jnp.bfloat16), v_h,
                            (((1,), (0,)), ((), ())),
                            preferred_element_type=jnp.float32,
                        ))
                    psend[b] = jnp.concatenate(ctx_heads, axis=1).astype(jnp.bfloat16)
                    lsend[b] = jnp.concatenate(l_cols, axis=1)

                for j in range(N_DEV - 1):
                    rp = pltpu.make_async_remote_copy(
                        src_ref=psend, dst_ref=pbuf.at[slot],
                        send_sem=send_sems.at[0, j], recv_sem=recv_sems.at[slot, 0],
                        device_id=((producer + j + 1) % N_DEV,),
                        device_id_type=pl.DeviceIdType.MESH,
                    )
                    rp.start()
                    rl = pltpu.make_async_remote_copy(
                        src_ref=lsend, dst_ref=lbuf.at[slot],
                        send_sem=send_sems.at[1, j], recv_sem=recv_sems.at[slot, 1],
                        device_id=((producer + j + 1) % N_DEV,),
                        device_id_type=pl.DeviceIdType.MESH,
                    )
                    rl.start()
                cp = pltpu.make_async_copy(psend, pbuf.at[slot], local_sems.at[0])
                cl = pltpu.make_async_copy(lsend, lbuf.at[slot], local_sems.at[1])
                cp.start()
                cl.start()
                cp.wait()
                cl.wait()

        for producer, slot in PRODUCERS:
            @pl.when(my_pos != producer)
            def _(producer=producer, slot=slot):
                rp = pltpu.make_async_remote_copy(
                    src_ref=psend, dst_ref=pbuf.at[slot],
                    send_sem=send_sems.at[0, 0], recv_sem=recv_sems.at[slot, 0],
                    device_id=(producer,), device_id_type=pl.DeviceIdType.MESH,
                )
                rp.wait_recv()
                rl = pltpu.make_async_remote_copy(
                    src_ref=lsend, dst_ref=lbuf.at[slot],
                    send_sem=send_sems.at[1, 0], recv_sem=recv_sems.at[slot, 1],
                    device_id=(producer,), device_id_type=pl.DeviceIdType.MESH,
                )
                rl.wait_recv()

        wo = wo_ref[...].astype(jnp.bfloat16)
        for b in range(B):
            ld = lbuf[0, b] + lbuf[1, b]
            ctx_heads = []
            for h in range(HQ):
                hs = slice(h * DH, (h + 1) * DH)
                num = (pbuf[0, b][:, hs].astype(jnp.float32)
                       + pbuf[1, b][:, hs].astype(jnp.float32))
                ctx_heads.append((num / ld[:, h:h + 1]).astype(jnp.bfloat16))
            ctx_b = jnp.concatenate(ctx_heads, axis=1)
            out_ref[b] = lax.dot_general(
                ctx_b, wo, (((1,), (0,)), ((), ())),
                preferred_element_type=jnp.float32,
            )

        for producer, slot in PRODUCERS:
            @pl.when(my_pos == producer)
            def _(producer=producer, slot=slot):
                for j in range(N_DEV - 1):
                    rp = pltpu.make_async_remote_copy(
                        src_ref=psend, dst_ref=pbuf.at[slot],
                        send_sem=send_sems.at[0, j], recv_sem=recv_sems.at[slot, 0],
                        device_id=((producer + j + 1) % N_DEV,),
                        device_id_type=pl.DeviceIdType.MESH,
                    )
                    rp.wait_send()
                    rl = pltpu.make_async_remote_copy(
                        src_ref=lsend, dst_ref=lbuf.at[slot],
                        send_sem=send_sems.at[1, j], recv_sem=recv_sems.at[slot, 1],
                        device_id=((producer + j + 1) % N_DEV,),
                        device_id_type=pl.DeviceIdType.MESH,
                    )
                    rl.wait_send()

    return pl.pallas_call(
        body,
        out_shape=jax.ShapeDtypeStruct((B, SQ, D_MODEL), jnp.float32),
        in_specs=[pl.BlockSpec(memory_space=pltpu.VMEM)] * 5,
        out_specs=pl.BlockSpec(memory_space=pltpu.VMEM),
        scratch_shapes=[
            pltpu.VMEM((B, SQ, D_QK), jnp.bfloat16),
            pltpu.VMEM((B, SQ, HQ), jnp.float32),
            pltpu.VMEM((2, B, SQ, D_QK), jnp.bfloat16),
            pltpu.VMEM((2, B, SQ, HQ), jnp.float32),
            pltpu.SemaphoreType.DMA((2, N_DEV - 1)),
            pltpu.SemaphoreType.DMA((2, 2)),
            pltpu.SemaphoreType.DMA((2,)),
        ],
        compiler_params=pltpu.CompilerParams(collective_id=0),
    )(x, Wq, K2, V2, Wo)


# device time: 14245 ns/iter; 1.1014x vs baseline; 1.1014x over previous
import jax
import jax.numpy as jnp
from jax import lax
from jax.experimental import pallas as pl
from jax.experimental.pallas import tpu as pltpu

N_DEV = 4
B, SQ, SKV, HQ, DH = 2, 128, 512, 4, 64
D_MODEL = 512
D_QK = HQ * DH
L_LANES = 16
D_PKT = D_QK + HQ * L_LANES
SKV_PER = SKV // N_DEV
PRODUCERS = ((0, 0), (2, 1))


def kernel(x, Wq, K_ext, V_ext, Wo):
    K2 = K_ext.reshape(B, SKV_PER, D_QK)
    V2 = V_ext.reshape(B, SKV_PER, D_QK)

    def body(x_ref, wq_ref, k_ref, v_ref, wo_ref, out_ref,
             psend, pbuf, send_sems, recv_sems, local_sems):
        my_pos = lax.axis_index("i")

        barrier_sem = pltpu.get_barrier_semaphore()
        for o in range(1, N_DEV):
            pl.semaphore_signal(
                barrier_sem, inc=1,
                device_id=((my_pos + o) % N_DEV,),
                device_id_type=pl.DeviceIdType.MESH,
            )
        pl.semaphore_wait(barrier_sem, N_DEV - 1)

        row = lax.broadcasted_iota(jnp.int32, (SQ, SKV_PER), 0)
        col = lax.broadcasted_iota(jnp.int32, (SQ, SKV_PER), 1)
        mask = (row // 64) == (col // 64)

        for producer, slot in PRODUCERS:
            @pl.when(my_pos == producer)
            def _(producer=producer, slot=slot):
                wq = wq_ref[...].astype(jnp.bfloat16)
                for b in range(B):
                    xb = x_ref[b].astype(jnp.bfloat16)
                    qb = lax.dot_general(
                        xb, wq, (((1,), (0,)), ((), ())),
                        preferred_element_type=jnp.float32,
                    )
                    ctx_heads = []
                    l_parts = []
                    for h in range(HQ):
                        hs = slice(h * DH, (h + 1) * DH)
                        q_h = qb[:, hs].astype(jnp.bfloat16)
                        k_h = k_ref[b][:, hs].astype(jnp.bfloat16)
                        scores = lax.dot_general(
                            q_h, k_h, (((1,), (1,)), ((), ())),
                            preferred_element_type=jnp.float32,
                        ) * 0.125
                        w = jnp.where(mask, jnp.exp(scores), 0.0)
                        l_h = jnp.sum(w, axis=1, keepdims=True)
                        l_parts.append(jnp.broadcast_to(l_h, (SQ, L_LANES)))
                        v_h = v_ref[b][:, hs].astype(jnp.bfloat16)
                        ctx_heads.append(lax.dot_general(
                            w.astype(jnp.bfloat16), v_h,
                            (((1,), (0,)), ((), ())),
                            preferred_element_type=jnp.float32,
                        ))
                    psend[b] = jnp.concatenate(
                        ctx_heads + l_parts, axis=1
                    ).astype(jnp.bfloat16)

                for j in range(N_DEV - 1):
                    rp = pltpu.make_async_remote_copy(
                        src_ref=psend, dst_ref=pbuf.at[slot],
                        send_sem=send_sems.at[j], recv_sem=recv_sems.at[slot],
                        device_id=((producer + j + 1) % N_DEV,),
                        device_id_type=pl.DeviceIdType.MESH,
                    )
                    rp.start()
                cp = pltpu.make_async_copy(psend, pbuf.at[slot], local_sems.at[0])
                cp.start()
                cp.wait()

        for producer, slot in PRODUCERS:
            @pl.when(my_pos != producer)
            def _(producer=producer, slot=slot):
                rp = pltpu.make_async_remote_copy(
                    src_ref=psend, dst_ref=pbuf.at[slot],
                    send_sem=send_sems.at[0], recv_sem=recv_sems.at[slot],
                    device_id=(producer,), device_id_type=pl.DeviceIdType.MESH,
                )
                rp.wait_recv()

        wo = wo_ref[...].astype(jnp.bfloat16)
        for b in range(B):
            p0 = pbuf[0, b]
            p1 = pbuf[1, b]
            ctx_heads = []
            for h in range(HQ):
                hs = slice(h * DH, (h + 1) * DH)
                lc = D_QK + h * L_LANES
                num = (p0[:, hs].astype(jnp.float32)
                       + p1[:, hs].astype(jnp.float32))
                ld = (p0[:, lc:lc + 1].astype(jnp.float32)
                      + p1[:, lc:lc + 1].astype(jnp.float32))
                ctx_heads.append((num / ld).astype(jnp.bfloat16))
            ctx_b = jnp.concatenate(ctx_heads, axis=1)
            out_ref[b] = lax.dot_general(
                ctx_b, wo, (((1,), (0,)), ((), ())),
                preferred_element_type=jnp.float32,
            )

        for producer, slot in PRODUCERS:
            @pl.when(my_pos == producer)
            def _(producer=producer, slot=slot):
                for j in range(N_DEV - 1):
                    rp = pltpu.make_async_remote_copy(
                        src_ref=psend, dst_ref=pbuf.at[slot],
                        send_sem=send_sems.at[j], recv_sem=recv_sems.at[slot],
                        device_id=((producer + j + 1) % N_DEV,),
                        device_id_type=pl.DeviceIdType.MESH,
                    )
                    rp.wait_send()

    return pl.pallas_call(
        body,
        out_shape=jax.ShapeDtypeStruct((B, SQ, D_MODEL), jnp.float32),
        in_specs=[pl.BlockSpec(memory_space=pltpu.VMEM)] * 5,
        out_specs=pl.BlockSpec(memory_space=pltpu.VMEM),
        scratch_shapes=[
            pltpu.VMEM((B, SQ, D_PKT), jnp.bfloat16),
            pltpu.VMEM((2, B, SQ, D_PKT), jnp.bfloat16),
            pltpu.SemaphoreType.DMA((N_DEV - 1,)),
            pltpu.SemaphoreType.DMA((2,)),
            pltpu.SemaphoreType.DMA((1,)),
        ],
        compiler_params=pltpu.CompilerParams(collective_id=0),
    )(x, Wq, K2, V2, Wo)


# device time: 14238 ns/iter; 1.1019x vs baseline; 1.0005x over previous
import jax
import jax.numpy as jnp
from jax import lax
from jax.experimental import pallas as pl
from jax.experimental.pallas import tpu as pltpu

N_DEV = 4
B, SQ, SKV, HQ, DH = 2, 128, 512, 4, 64
D_MODEL = 512
D_QK = HQ * DH
L_LANES = 8
D_PKT = D_QK + HQ * L_LANES
SKV_PER = SKV // N_DEV
PRODUCERS = ((0, 0), (2, 1))


def kernel(x, Wq, K_ext, V_ext, Wo):
    K2 = K_ext.reshape(B, SKV_PER, D_QK)
    V2 = V_ext.reshape(B, SKV_PER, D_QK)

    def body(x_ref, wq_ref, k_ref, v_ref, wo_ref, out_ref,
             psend, pbuf, send_sems, recv_sems, local_sems):
        my_pos = lax.axis_index("i")

        row = lax.broadcasted_iota(jnp.int32, (SQ, SKV_PER), 0)
        col = lax.broadcasted_iota(jnp.int32, (SQ, SKV_PER), 1)
        mask = (row // 64) == (col // 64)

        for producer, slot in PRODUCERS:
            @pl.when(my_pos == producer)
            def _(producer=producer, slot=slot):
                wq = wq_ref[...].astype(jnp.bfloat16)
                for b in range(B):
                    xb = x_ref[b].astype(jnp.bfloat16)
                    qb = lax.dot_general(
                        xb, wq, (((1,), (0,)), ((), ())),
                        preferred_element_type=jnp.float32,
                    )
                    ctx_heads = []
                    l_parts = []
                    for h in range(HQ):
                        hs = slice(h * DH, (h + 1) * DH)
                        q_h = qb[:, hs].astype(jnp.bfloat16)
                        k_h = k_ref[b][:, hs].astype(jnp.bfloat16)
                        scores = lax.dot_general(
                            q_h, k_h, (((1,), (1,)), ((), ())),
                            preferred_element_type=jnp.float32,
                        ) * 0.125
                        w = jnp.where(mask, jnp.exp(scores), 0.0)
                        l_h = jnp.sum(w, axis=1, keepdims=True)
                        l_parts.append(jnp.broadcast_to(l_h, (SQ, L_LANES)))
                        v_h = v_ref[b][:, hs].astype(jnp.bfloat16)
                        ctx_heads.append(lax.dot_general(
                            w.astype(jnp.bfloat16), v_h,
                            (((1,), (0,)), ((), ())),
                            preferred_element_type=jnp.float32,
                        ))
                    psend[b] = jnp.concatenate(
                        ctx_heads + l_parts, axis=1
                    ).astype(jnp.bfloat16)

        credit_sem = pltpu.get_barrier_semaphore()
        for producer, _slot in PRODUCERS:
            @pl.when(my_pos != producer)
            def _(producer=producer):
                pl.semaphore_signal(
                    credit_sem, inc=1,
                    device_id=(producer,),
                    device_id_type=pl.DeviceIdType.MESH,
                )

        for producer, slot in PRODUCERS:
            @pl.when(my_pos == producer)
            def _(producer=producer, slot=slot):
                pl.semaphore_wait(credit_sem, N_DEV - 1)
                for j in range(N_DEV - 1):
                    rp = pltpu.make_async_remote_copy(
                        src_ref=psend, dst_ref=pbuf.at[slot],
                        send_sem=send_sems.at[j], recv_sem=recv_sems.at[slot],
                        device_id=((producer + j + 1) % N_DEV,),
                        device_id_type=pl.DeviceIdType.MESH,
                    )
                    rp.start()
                cp = pltpu.make_async_copy(psend, pbuf.at[slot], local_sems.at[0])
                cp.start()
                cp.wait()

        for producer, slot in PRODUCERS:
            @pl.when(my_pos != producer)
            def _(producer=producer, slot=slot):
                rp = pltpu.make_async_remote_copy(
                    src_ref=psend, dst_ref=pbuf.at[slot],
                    send_sem=send_sems.at[0], recv_sem=recv_sems.at[slot],
                    device_id=(producer,), device_id_type=pl.DeviceIdType.MESH,
                )
                rp.wait_recv()

        wo = wo_ref[...].astype(jnp.bfloat16)
        for b in range(B):
            p0 = pbuf[0, b]
            p1 = pbuf[1, b]
            ctx_heads = []
            for h in range(HQ):
                hs = slice(h * DH, (h + 1) * DH)
                lc = D_QK + h * L_LANES
                num = (p0[:, hs].astype(jnp.float32)
                       + p1[:, hs].astype(jnp.float32))
                ld = (p0[:, lc:lc + 1].astype(jnp.float32)
                      + p1[:, lc:lc + 1].astype(jnp.float32))
                ctx_heads.append((num / ld).astype(jnp.bfloat16))
            ctx_b = jnp.concatenate(ctx_heads, axis=1)
            out_ref[b] = lax.dot_general(
                ctx_b, wo, (((1,), (0,)), ((), ())),
                preferred_element_type=jnp.float32,
            )

        for producer, slot in PRODUCERS:
            @pl.when(my_pos == producer)
            def _(producer=producer, slot=slot):
                for j in range(N_DEV - 1):
                    rp = pltpu.make_async_remote_copy(
                        src_ref=psend, dst_ref=pbuf.at[slot],
                        send_sem=send_sems.at[j], recv_sem=recv_sems.at[slot],
                        device_id=((producer + j + 1) % N_DEV,),
                        device_id_type=pl.DeviceIdType.MESH,
                    )
                    rp.wait_send()

    return pl.pallas_call(
        body,
        out_shape=jax.ShapeDtypeStruct((B, SQ, D_MODEL), jnp.float32),
        in_specs=[pl.BlockSpec(memory_space=pltpu.VMEM)] * 5,
        out_specs=pl.BlockSpec(memory_space=pltpu.VMEM),
        scratch_shapes=[
            pltpu.VMEM((B, SQ, D_PKT), jnp.bfloat16),
            pltpu.VMEM((2, B, SQ, D_PKT), jnp.bfloat16),
            pltpu.SemaphoreType.DMA((N_DEV - 1,)),
            pltpu.SemaphoreType.DMA((2,)),
            pltpu.SemaphoreType.DMA((1,)),
        ],
        compiler_params=pltpu.CompilerParams(collective_id=0),
    )(x, Wq, K2, V2, Wo)


# device time: 13960 ns/iter; 1.1239x vs baseline; 1.0199x over previous
import jax
import jax.numpy as jnp
from jax import lax
from jax.experimental import pallas as pl
from jax.experimental.pallas import tpu as pltpu

N_DEV = 4
B, SQ, SKV, HQ, DH = 2, 128, 512, 4, 64
D_MODEL = 512
D_QK = HQ * DH
L_LANES = 8
D_PKT = D_QK + HQ * L_LANES
SKV_PER = SKV // N_DEV
PRODUCERS = ((0, 0), (2, 1))


def kernel(x, Wq, K_ext, V_ext, Wo):
    K2 = K_ext.reshape(B, SKV_PER, D_QK)
    V2 = V_ext.reshape(B, SKV_PER, D_QK)

    def body(x_ref, wq_ref, k_ref, v_ref, wo_ref, out_ref,
             psend, pbuf, send_sems, recv_sems, local_sems):
        my_pos = lax.axis_index("i")

        credit_sem = pltpu.get_barrier_semaphore()
        for producer, _slot in PRODUCERS:
            @pl.when(my_pos != producer)
            def _(producer=producer):
                pl.semaphore_signal(
                    credit_sem, inc=1,
                    device_id=(producer,),
                    device_id_type=pl.DeviceIdType.MESH,
                )

        def partial_for_batch(b):
            wq = wq_ref[...].astype(jnp.bfloat16)
            xb = x_ref[b].astype(jnp.bfloat16)
            qb = lax.dot_general(
                xb, wq, (((1,), (0,)), ((), ())),
                preferred_element_type=jnp.float32,
            )
            ctx_heads = []
            l_parts = []
            for h in range(HQ):
                hs = slice(h * DH, (h + 1) * DH)
                ctx_blocks = []
                l_blocks = []
                for blk in range(2):
                    rs = slice(blk * 64, (blk + 1) * 64)
                    q_b = qb[rs, hs].astype(jnp.bfloat16)
                    k_b = k_ref[b][rs, hs].astype(jnp.bfloat16)
                    scores = lax.dot_general(
                        q_b, k_b, (((1,), (1,)), ((), ())),
                        preferred_element_type=jnp.float32,
                    ) * 0.125
                    w = jnp.exp(scores)
                    l_blocks.append(jnp.sum(w, axis=1, keepdims=True))
                    v_b = v_ref[b][rs, hs].astype(jnp.bfloat16)
                    ctx_blocks.append(lax.dot_general(
                        w.astype(jnp.bfloat16), v_b,
                        (((1,), (0,)), ((), ())),
                        preferred_element_type=jnp.float32,
                    ))
                ctx_heads.append(jnp.concatenate(ctx_blocks, axis=0))
                l_h = jnp.concatenate(l_blocks, axis=0)
                l_parts.append(jnp.broadcast_to(l_h, (SQ, L_LANES)))
            return jnp.concatenate(ctx_heads + l_parts, axis=1).astype(jnp.bfloat16)

        def sends_for_batch(producer, slot, b, start):
            for j in range(N_DEV - 1):
                r = pltpu.make_async_remote_copy(
                    src_ref=psend.at[b], dst_ref=pbuf.at[slot, b],
                    send_sem=send_sems.at[j, b], recv_sem=recv_sems.at[slot, b],
                    device_id=((producer + j + 1) % N_DEV,),
                    device_id_type=pl.DeviceIdType.MESH,
                )
                if start:
                    r.start()
                else:
                    r.wait_send()

        for producer, slot in PRODUCERS:
            @pl.when(my_pos == producer)
            def _(producer=producer, slot=slot):
                psend[0] = partial_for_batch(0)
                pl.semaphore_wait(credit_sem, N_DEV - 1)
                sends_for_batch(producer, slot, 0, start=True)
                psend[1] = partial_for_batch(1)
                sends_for_batch(producer, slot, 1, start=True)
                cp = pltpu.make_async_copy(psend, pbuf.at[slot], local_sems.at[0])
                cp.start()
                cp.wait()

        for producer, slot in PRODUCERS:
            @pl.when(my_pos != producer)
            def _(producer=producer, slot=slot):
                for b in range(B):
                    r = pltpu.make_async_remote_copy(
                        src_ref=psend.at[b], dst_ref=pbuf.at[slot, b],
                        send_sem=send_sems.at[0, b], recv_sem=recv_sems.at[slot, b],
                        device_id=(producer,), device_id_type=pl.DeviceIdType.MESH,
                    )
                    r.wait_recv()

        wo = wo_ref[...].astype(jnp.bfloat16)
        for b in range(B):
            p0 = pbuf[0, b]
            p1 = pbuf[1, b]
            ctx_heads = []
            for h in range(HQ):
                hs = slice(h * DH, (h + 1) * DH)
                lc = D_QK + h * L_LANES
                num = (p0[:, hs].astype(jnp.float32)
                       + p1[:, hs].astype(jnp.float32))
                ld = (p0[:, lc:lc + 1].astype(jnp.float32)
                      + p1[:, lc:lc + 1].astype(jnp.float32))
                ctx_heads.append((num / ld).astype(jnp.bfloat16))
            ctx_b = jnp.concatenate(ctx_heads, axis=1)
            out_ref[b] = lax.dot_general(
                ctx_b, wo, (((1,), (0,)), ((), ())),
                preferred_element_type=jnp.float32,
            )

        for producer, slot in PRODUCERS:
            @pl.when(my_pos == producer)
            def _(producer=producer, slot=slot):
                for b in range(B):
                    sends_for_batch(producer, slot, b, start=False)

    return pl.pallas_call(
        body,
        out_shape=jax.ShapeDtypeStruct((B, SQ, D_MODEL), jnp.float32),
        in_specs=[pl.BlockSpec(memory_space=pltpu.VMEM)] * 5,
        out_specs=pl.BlockSpec(memory_space=pltpu.VMEM),
        scratch_shapes=[
            pltpu.VMEM((B, SQ, D_PKT), jnp.bfloat16),
            pltpu.VMEM((2, B, SQ, D_PKT), jnp.bfloat16),
            pltpu.SemaphoreType.DMA((N_DEV - 1, B)),
            pltpu.SemaphoreType.DMA((2, B)),
            pltpu.SemaphoreType.DMA((1,)),
        ],
        compiler_params=pltpu.CompilerParams(collective_id=0),
    )(x, Wq, K2, V2, Wo)


# device time: 13841 ns/iter; 1.1335x vs baseline; 1.0086x over previous
import jax
import jax.numpy as jnp
from jax import lax
from jax.experimental import pallas as pl
from jax.experimental.pallas import tpu as pltpu

N_DEV = 4
B, SQ, SKV, HQ, DH = 2, 128, 512, 4, 64
D_MODEL = 512
D_QK = HQ * DH
L_LANES = 8
D_PKT = D_QK + HQ * L_LANES
SKV_PER = SKV // N_DEV
PRODUCERS = ((0, 0), (2, 1))


def kernel(x, Wq, K_ext, V_ext, Wo):
    K2 = K_ext.reshape(B, SKV_PER, D_QK)
    V2 = V_ext.reshape(B, SKV_PER, D_QK)

    def body(x_ref, wq_ref, k_ref, v_ref, wo_ref, out_ref,
             psend, pbuf, send_sems, recv_sems, local_sems):
        my_pos = lax.axis_index("i")

        credit_sem = pltpu.get_barrier_semaphore()
        for producer, _slot in PRODUCERS:
            @pl.when(my_pos != producer)
            def _(producer=producer):
                pl.semaphore_signal(
                    credit_sem, inc=1,
                    device_id=(producer,),
                    device_id_type=pl.DeviceIdType.MESH,
                )

        def partial_for_batch(b):
            wq = wq_ref[...].astype(jnp.bfloat16)
            xb = x_ref[b].astype(jnp.bfloat16)
            qb = lax.dot_general(
                xb, wq, (((1,), (0,)), ((), ())),
                preferred_element_type=jnp.float32,
            )
            ctx_heads = []
            l_parts = []
            for h in range(HQ):
                hs = slice(h * DH, (h + 1) * DH)
                ctx_blocks = []
                l_blocks = []
                for blk in range(2):
                    rs = slice(blk * 64, (blk + 1) * 64)
                    q_b = qb[rs, hs].astype(jnp.bfloat16)
                    k_b = k_ref[b][rs, hs].astype(jnp.bfloat16)
                    scores = lax.dot_general(
                        q_b, k_b, (((1,), (1,)), ((), ())),
                        preferred_element_type=jnp.float32,
                    ) * 0.125
                    w = jnp.exp(scores)
                    l_blocks.append(jnp.sum(w, axis=1, keepdims=True))
                    v_b = v_ref[b][rs, hs].astype(jnp.bfloat16)
                    ctx_blocks.append(lax.dot_general(
                        w.astype(jnp.bfloat16), v_b,
                        (((1,), (0,)), ((), ())),
                        preferred_element_type=jnp.float32,
                    ))
                ctx_heads.append(jnp.concatenate(ctx_blocks, axis=0))
                l_h = jnp.concatenate(l_blocks, axis=0)
                l_parts.append(jnp.broadcast_to(l_h, (SQ, L_LANES)))
            return jnp.concatenate(ctx_heads + l_parts, axis=1).astype(jnp.bfloat16)

        def sends_for_batch(producer, slot, b, start):
            for j in range(N_DEV - 1):
                r = pltpu.make_async_remote_copy(
                    src_ref=psend.at[b], dst_ref=pbuf.at[slot, b],
                    send_sem=send_sems.at[j, b], recv_sem=recv_sems.at[slot, b],
                    device_id=((producer + j + 1) % N_DEV,),
                    device_id_type=pl.DeviceIdType.MESH,
                )
                if start:
                    r.start()
                else:
                    r.wait_send()

        for producer, slot in PRODUCERS:
            @pl.when(my_pos == producer)
            def _(producer=producer, slot=slot):
                psend[0] = partial_for_batch(0)
                pl.semaphore_wait(credit_sem, N_DEV - 1)
                sends_for_batch(producer, slot, 0, start=True)
                psend[1] = partial_for_batch(1)
                sends_for_batch(producer, slot, 1, start=True)
                cp = pltpu.make_async_copy(psend, pbuf.at[slot], local_sems.at[0])
                cp.start()
                cp.wait()

        wo = wo_ref[...].astype(jnp.bfloat16)
        for b in range(B):
            for producer, slot in PRODUCERS:
                @pl.when(my_pos != producer)
                def _(producer=producer, slot=slot, b=b):
                    r = pltpu.make_async_remote_copy(
                        src_ref=psend.at[b], dst_ref=pbuf.at[slot, b],
                        send_sem=send_sems.at[0, b], recv_sem=recv_sems.at[slot, b],
                        device_id=(producer,), device_id_type=pl.DeviceIdType.MESH,
                    )
                    r.wait_recv()
            p0 = pbuf[0, b]
            p1 = pbuf[1, b]
            ctx_heads = []
            for h in range(HQ):
                hs = slice(h * DH, (h + 1) * DH)
                lc = D_QK + h * L_LANES
                num = (p0[:, hs].astype(jnp.float32)
                       + p1[:, hs].astype(jnp.float32))
                ld = (p0[:, lc:lc + 1].astype(jnp.float32)
                      + p1[:, lc:lc + 1].astype(jnp.float32))
                ctx_heads.append((num / ld).astype(jnp.bfloat16))
            ctx_b = jnp.concatenate(ctx_heads, axis=1)
            out_ref[b] = lax.dot_general(
                ctx_b, wo, (((1,), (0,)), ((), ())),
                preferred_element_type=jnp.float32,
            )

        for producer, slot in PRODUCERS:
            @pl.when(my_pos == producer)
            def _(producer=producer, slot=slot):
                for b in range(B):
                    sends_for_batch(producer, slot, b, start=False)

    return pl.pallas_call(
        body,
        out_shape=jax.ShapeDtypeStruct((B, SQ, D_MODEL), jnp.float32),
        in_specs=[pl.BlockSpec(memory_space=pltpu.VMEM)] * 5,
        out_specs=pl.BlockSpec(memory_space=pltpu.VMEM),
        scratch_shapes=[
            pltpu.VMEM((B, SQ, D_PKT), jnp.bfloat16),
            pltpu.VMEM((2, B, SQ, D_PKT), jnp.bfloat16),
            pltpu.SemaphoreType.DMA((N_DEV - 1, B)),
            pltpu.SemaphoreType.DMA((2, B)),
            pltpu.SemaphoreType.DMA((1,)),
        ],
        compiler_params=pltpu.CompilerParams(collective_id=0),
    )(x, Wq, K2, V2, Wo)
